# baseline (speedup 1.0000x reference)
import functools

import jax
import jax.numpy as jnp
from jax.experimental import pallas as pl
from jax.experimental.pallas import tpu as pltpu

N = 10000
E = 320000
HEADS = 4
GAT_OUT = 256
N_GRAPHS = 64
FUSED = 512


def _fc_kernel(p_ref, w_ref, b_ref, o_ref):
    o_ref[...] = jax.nn.relu(
        jnp.dot(p_ref[...], w_ref[...], preferred_element_type=jnp.float32)
        + b_ref[...]
    )


def _gcn_conv(x, src, dst, W, b):
    h = x @ W
    deg = jnp.zeros((N,), x.dtype).at[dst].add(1.0)
    dinv = jnp.where(deg > 0, 1.0 / jnp.sqrt(deg), 0.0)
    norm = dinv[src] * dinv[dst]
    out = jnp.zeros((N, W.shape[1]), x.dtype).at[dst].add(h[src] * norm[:, None])
    return out + b


def _bn(x, gamma, beta):
    m = jnp.mean(x, axis=0)
    v = jnp.var(x, axis=0)
    return (x - m) / jnp.sqrt(v + 1e-5) * gamma + beta


def _gat_conv(x, src, dst, Wg, att_src, att_dst, bg):
    h = (x @ Wg).reshape(N, HEADS, GAT_OUT)
    a_s = jnp.sum(h * att_src, axis=-1)
    a_d = jnp.sum(h * att_dst, axis=-1)
    e = a_s[src] + a_d[dst]
    e = jax.nn.leaky_relu(e, 0.2)
    emax = jnp.full((N, HEADS), -1e30, x.dtype).at[dst].max(e)
    ex = jnp.exp(e - emax[dst])
    den = jnp.zeros((N, HEADS), x.dtype).at[dst].add(ex)
    coef = ex / (den[dst] + 1e-16)
    out = jnp.zeros((N, HEADS, GAT_OUT), x.dtype).at[dst].add(h[src] * coef[:, :, None])
    return jnp.mean(out, axis=1) + bg


def kernel(x, edge_index, batch, W1, b1, g1, be1, W2, b2, g2, be2,
           W3, b3, g3, be3, Wg, att_src, att_dst, bg, Wfc, bfc):
    loop = jnp.arange(N, dtype=edge_index.dtype)
    src = jnp.concatenate([edge_index[0], loop])
    dst = jnp.concatenate([edge_index[1], loop])
    h = _gcn_conv(x, src, dst, W1, b1)
    h = jax.nn.relu(_bn(h, g1, be1))
    h = _gcn_conv(h, src, dst, W2, b2)
    h = jax.nn.relu(_bn(h, g2, be2))
    h = _gcn_conv(h, src, dst, W3, b3)
    h = jax.nn.relu(_bn(h, g3, be3))
    h = _gat_conv(h, src, dst, Wg, att_src, att_dst, bg)
    h = jax.nn.relu(_bn(h, g3, be3))
    sums = jax.ops.segment_sum(h, batch, num_segments=N_GRAPHS)
    cnt = jax.ops.segment_sum(jnp.ones((N,), h.dtype), batch, num_segments=N_GRAPHS)
    pooled = sums / jnp.clip(cnt, 1.0)[:, None]
    out = pl.pallas_call(
        _fc_kernel,
        out_shape=jax.ShapeDtypeStruct((N_GRAPHS, FUSED), jnp.float32),
    )(pooled, Wfc, bfc.reshape(1, FUSED))
    return out


# speedup vs baseline: 8.0219x; 8.0219x over previous
import functools

import jax
import jax.numpy as jnp
from jax import lax
from jax.experimental import pallas as pl
from jax.experimental.pallas import tpu as pltpu
from jax.experimental.pallas import tpu_sc as plsc

N = 10000
NPAD = 10240
NPT = 320
NTILES = 32
E = 320000
ETOT = E + N
ECHUNK = 2048
EPAD = ((ETOT + ECHUNK - 1) // ECHUNK) * ECHUNK
NCHUNKS = EPAD // ECHUNK
CAP = EPAD + 2 * ECHUNK
HEADS = 4
GAT_OUT = 256
N_GRAPHS = 64
FUSED = 512
SENTINEL = 1 << 20

_MESH = plsc.VectorSubcoreMesh(core_axis_name="c", subcore_axis_name="s")
_SC_PARAMS = pltpu.CompilerParams(needs_layout_passes=False)


def _wid_base():
    wid = lax.axis_index("s") * 2 + lax.axis_index("c")
    return wid, pl.multiple_of(wid * NPT, 8)


def _zero_1d(ref, nwords):
    z = jnp.zeros((16,), jnp.float32)

    def body(i, _):
        ref[pl.ds(i * 16, 16)] = z
        return 0

    lax.fori_loop(0, nwords // 16, body, 0)


def _zero_rows(ref, nrows, ncols):
    z = jnp.zeros((16,), jnp.float32)

    def body(r, _):
        for j in range(ncols // 16):
            ref[r, pl.ds(j * 16, 16)] = z
        return 0

    lax.fori_loop(0, nrows, body, 0)


@functools.partial(
    pl.kernel,
    out_type=(
        jax.ShapeDtypeStruct((NPAD,), jnp.float32),
        jax.ShapeDtypeStruct((NTILES * CAP,), jnp.int32),
        jax.ShapeDtypeStruct((NTILES * CAP,), jnp.int32),
        jax.ShapeDtypeStruct((NTILES * 16,), jnp.int32),
    ),
    mesh=_MESH,
    compiler_params=_SC_PARAMS,
    scratch_types=[
        pltpu.VMEM((ECHUNK,), jnp.int32),
        pltpu.VMEM((ECHUNK,), jnp.int32),
        pltpu.VMEM((2 * ECHUNK + 16,), jnp.int32),
        pltpu.VMEM((2 * ECHUNK + 16,), jnp.int32),
        pltpu.VMEM((NPT,), jnp.float32),
        pltpu.VMEM((16,), jnp.int32),
    ],
)
def _pre_kernel(src_h, dst_h, deg_h, bsrc_h, bdst_h, cnt_h,
                esrc, edst, osrc, odst, degl, splat):
    wid, base = _wid_base()
    izero = jnp.zeros((16,), jnp.int32)

    def zi(ref, nwords):
        def body(i, _):
            ref[pl.ds(i * 16, 16)] = izero
            return 0
        lax.fori_loop(0, nwords // 16, body, 0)

    zi(osrc, 2 * ECHUNK + 16)
    zi(odst, 2 * ECHUNK + 16)
    _zero_1d(degl, NPT)

    ones = jnp.ones((16,), jnp.float32)

    def chunk(k, carry):
        cursor, out_off = carry
        eoff = pl.multiple_of(k * ECHUNK, 8)
        pltpu.sync_copy(src_h.at[pl.ds(eoff, ECHUNK)], esrc)
        pltpu.sync_copy(dst_h.at[pl.ds(eoff, ECHUNK)], edst)

        def inner(i, cur):
            sv = esrc[pl.ds(i * 16, 16)]
            dv = edst[pl.ds(i * 16, 16)]
            mask = (dv >= base) & (dv < base + NPT)
            dloc = jnp.where(mask, dv - base, 0)
            plsc.addupdate_scatter(degl, [dloc], ones, mask=mask)
            plsc.store_compressed(osrc.at[pl.ds(cur, 16)], sv, mask=mask)
            plsc.store_compressed(odst.at[pl.ds(cur, 16)], dloc, mask=mask)
            return cur + jnp.sum(jnp.where(mask, 1, 0))

        cursor = lax.fori_loop(0, ECHUNK // 16, inner, cursor)

        do_flush = cursor >= ECHUNK

        @pl.when(do_flush)
        def _():
            dsthb = pl.multiple_of(wid * CAP + out_off, 8)
            pltpu.sync_copy(osrc.at[pl.ds(0, ECHUNK)],
                            bsrc_h.at[pl.ds(dsthb, ECHUNK)])
            pltpu.sync_copy(odst.at[pl.ds(0, ECHUNK)],
                            bdst_h.at[pl.ds(dsthb, ECHUNK)])
            for r in range(ECHUNK // 16 + 1):
                osrc[pl.ds(r * 16, 16)] = osrc[pl.ds(ECHUNK + r * 16, 16)]
                odst[pl.ds(r * 16, 16)] = odst[pl.ds(ECHUNK + r * 16, 16)]

        shift = jnp.where(do_flush, ECHUNK, 0)
        return cursor - shift, out_off + shift

    cursor, out_off = lax.fori_loop(0, NCHUNKS, chunk, (jnp.int32(0), jnp.int32(0)))

    dsthb = pl.multiple_of(wid * CAP + out_off, 8)
    pltpu.sync_copy(osrc.at[pl.ds(0, ECHUNK)], bsrc_h.at[pl.ds(dsthb, ECHUNK)])
    pltpu.sync_copy(odst.at[pl.ds(0, ECHUNK)], bdst_h.at[pl.ds(dsthb, ECHUNK)])
    pltpu.sync_copy(osrc.at[pl.ds(ECHUNK, ECHUNK)],
                    bsrc_h.at[pl.ds(dsthb + ECHUNK, ECHUNK)])
    pltpu.sync_copy(odst.at[pl.ds(ECHUNK, ECHUNK)],
                    bdst_h.at[pl.ds(dsthb + ECHUNK, ECHUNK)])

    pltpu.sync_copy(degl, deg_h.at[pl.ds(base, NPT)])
    total = out_off + cursor
    splat[pl.ds(0, 16)] = jnp.broadcast_to(total, (16,)).astype(jnp.int32)
    pltpu.sync_copy(splat, cnt_h.at[pl.ds(pl.multiple_of(wid * 16, 8), 16)])


def _make_gcn_scatter(D, C=64):
    @functools.partial(
        pl.kernel,
        out_type=jax.ShapeDtypeStruct((NPAD, D), jnp.float32),
        mesh=_MESH,
    compiler_params=_SC_PARAMS,
        scratch_types=[
            pltpu.VMEM((NPAD,), jnp.float32),
            pltpu.VMEM((C,), jnp.int32),
            pltpu.VMEM((C + 16,), jnp.int32),
            pltpu.VMEM((C + 16,), jnp.float32),
            pltpu.VMEM((C, D), jnp.float32),
            pltpu.VMEM((NPT, D), jnp.float32),
            pltpu.VMEM((16,), jnp.int32),
            pltpu.SemaphoreType.DMA,
        ],
    )
    def gcn_k(y_h, dinv_h, bsrc_h, bdst_h, cnt_h, out_h,
              dinv_v, srcb, dstb, normb, rowbuf, acc, cntv, sem):
        wid, base = _wid_base()
        pltpu.sync_copy(cnt_h.at[pl.ds(pl.multiple_of(wid * 16, 8), 16)], cntv)
        c = cntv[pl.ds(0, 16)][0]
        pltpu.sync_copy(dinv_h, dinv_v)
        _zero_rows(acc, NPT, D)

        nch = (c + (C - 1)) // C

        def chunk(k, _):
            off = pl.multiple_of(wid * CAP + k * C, 8)
            pltpu.sync_copy(bsrc_h.at[pl.ds(off, C)], srcb)
            pltpu.sync_copy(bdst_h.at[pl.ds(off, C)], dstb.at[pl.ds(0, C)])
            pltpu.async_copy(y_h.at[srcb], rowbuf, sem).wait()
            for i in range(C // 16):
                sv = srcb[pl.ds(i * 16, 16)]
                dlv = dstb[pl.ds(i * 16, 16)]
                nm = (plsc.load_gather(dinv_v, [sv])
                      * plsc.load_gather(dinv_v, [dlv + base]))
                normb[pl.ds(i * 16, 16)] = nm

            cnt_k = jnp.minimum(c - k * C, C)

            def edge(e, _):
                dloc = dstb[pl.ds(e, 16)][0]
                nr = normb[pl.ds(e, 16)][0]
                for j in range(D // 16):
                    plsc.addupdate(acc.at[dloc, pl.ds(j * 16, 16)],
                                   nr * rowbuf[e, pl.ds(j * 16, 16)])
                return 0

            lax.fori_loop(0, cnt_k, edge, 0)
            return 0

        lax.fori_loop(0, nch, chunk, 0)
        pltpu.sync_copy(acc, out_h.at[pl.ds(base, NPT)])

    return gcn_k


_gcn_scatter_128 = _make_gcn_scatter(128)
_gcn_scatter_256 = _make_gcn_scatter(256)


CA = 512


@functools.partial(
    pl.kernel,
    out_type=(
        jax.ShapeDtypeStruct((HEADS * NPAD,), jnp.float32),
        jax.ShapeDtypeStruct((NTILES * HEADS * CAP,), jnp.float32),
    ),
    mesh=_MESH,
    compiler_params=_SC_PARAMS,
    scratch_types=[
        pltpu.VMEM((HEADS * NPAD,), jnp.float32),
        pltpu.VMEM((HEADS * NPT,), jnp.float32),
        pltpu.VMEM((HEADS * NPT,), jnp.float32),
        pltpu.VMEM((HEADS * CA,), jnp.float32),
        pltpu.VMEM((CA,), jnp.int32),
        pltpu.VMEM((CA,), jnp.int32),
        pltpu.VMEM((16,), jnp.int32),
    ],
)
def _gat_a_kernel(as_h, ad_h, bsrc_h, bdst_h, cnt_h, den_h, exb_h,
                  asv, adl, denl, exst, srcb, dstb, cntv):
    wid, base = _wid_base()
    pltpu.sync_copy(cnt_h.at[pl.ds(pl.multiple_of(wid * 16, 8), 16)], cntv)
    c = cntv[pl.ds(0, 16)][0]
    pltpu.sync_copy(as_h, asv)
    for h in range(HEADS):
        pltpu.sync_copy(ad_h.at[pl.ds(pl.multiple_of(h * NPAD + base, 8), NPT)],
                        adl.at[pl.ds(h * NPT, NPT)])
    _zero_1d(denl, HEADS * NPT)

    lanes = lax.iota(jnp.int32, 16)
    nch = (c + (CA - 1)) // CA

    def chunk(k, _):
        off = pl.multiple_of(wid * CAP + k * CA, 8)
        pltpu.sync_copy(bsrc_h.at[pl.ds(off, CA)], srcb)
        pltpu.sync_copy(bdst_h.at[pl.ds(off, CA)], dstb)
        for i in range(CA // 16):
            sv = srcb[pl.ds(i * 16, 16)]
            dlv = dstb[pl.ds(i * 16, 16)]
            mask = (k * CA + i * 16 + lanes) < c
            for h in range(HEADS):
                a1 = plsc.load_gather(asv, [sv + h * NPAD])
                a2 = plsc.load_gather(adl, [dlv + h * NPT])
                ev = a1 + a2
                ev = jnp.maximum(ev, 0.2 * ev)
                ex = jnp.exp(ev)
                plsc.addupdate_scatter(denl, [dlv + h * NPT], ex, mask=mask)
                exst[pl.ds(h * CA + i * 16, 16)] = ex
        for h in range(HEADS):
            pltpu.sync_copy(
                exst.at[pl.ds(h * CA, CA)],
                exb_h.at[pl.ds(pl.multiple_of((wid * HEADS + h) * CAP + k * CA, 8), CA)])
        return 0

    lax.fori_loop(0, nch, chunk, 0)
    for h in range(HEADS):
        pltpu.sync_copy(denl.at[pl.ds(h * NPT, NPT)],
                        den_h.at[pl.ds(pl.multiple_of(h * NPAD + base, 8), NPT)])


CB = 16


@functools.partial(
    pl.kernel,
    out_type=jax.ShapeDtypeStruct((NPAD, GAT_OUT), jnp.float32),
    mesh=_MESH,
    compiler_params=_SC_PARAMS,
    scratch_types=[
        pltpu.VMEM((HEADS * NPT,), jnp.float32),
        pltpu.VMEM((NPT, GAT_OUT), jnp.float32),
        pltpu.VMEM((CB, HEADS * GAT_OUT), jnp.float32),
        pltpu.VMEM((HEADS * CB,), jnp.float32),
        pltpu.VMEM((HEADS * CB + 16,), jnp.float32),
        pltpu.VMEM((CB,), jnp.int32),
        pltpu.VMEM((CB + 16,), jnp.int32),
        pltpu.VMEM((16,), jnp.int32),
        pltpu.SemaphoreType.DMA,
    ],
)
def _gat_b_kernel(hg_h, den_h, exb_h, bsrc_h, bdst_h, cnt_h, out_h,
                  denl, acc, rowbuf, exc, coefb, srcb, dstb, cntv, sem):
    wid, base = _wid_base()
    pltpu.sync_copy(cnt_h.at[pl.ds(pl.multiple_of(wid * 16, 8), 16)], cntv)
    c = cntv[pl.ds(0, 16)][0]
    for h in range(HEADS):
        pltpu.sync_copy(den_h.at[pl.ds(pl.multiple_of(h * NPAD + base, 8), NPT)],
                        denl.at[pl.ds(h * NPT, NPT)])
    _zero_rows(acc, NPT, GAT_OUT)

    nch = (c + (CB - 1)) // CB

    def chunk(k, _):
        off = pl.multiple_of(wid * CAP + k * CB, 8)
        pltpu.sync_copy(bsrc_h.at[pl.ds(off, CB)], srcb)
        pltpu.sync_copy(bdst_h.at[pl.ds(off, CB)], dstb.at[pl.ds(0, CB)])
        cp = pltpu.async_copy(hg_h.at[srcb], rowbuf, sem)
        for h in range(HEADS):
            pltpu.sync_copy(
                exb_h.at[pl.ds(pl.multiple_of((wid * HEADS + h) * CAP + k * CB, 8), CB)],
                exc.at[pl.ds(h * CB, CB)])
        dlv = dstb[pl.ds(0, 16)]
        for h in range(HEADS):
            dg = plsc.load_gather(denl, [dlv + h * NPT])
            coefb[pl.ds(h * CB, 16)] = (
                exc[pl.ds(h * CB, 16)] / (dg + 1e-16) * 0.25)
        cp.wait()

        cnt_k = jnp.minimum(c - k * CB, CB)

        def edge(e, _):
            dloc = dstb[pl.ds(e, 16)][0]
            c0 = coefb[pl.ds(e, 16)][0]
            c1 = coefb[pl.ds(CB + e, 16)][0]
            c2 = coefb[pl.ds(2 * CB + e, 16)][0]
            c3 = coefb[pl.ds(3 * CB + e, 16)][0]
            for j in range(GAT_OUT // 16):
                v = (c0 * rowbuf[e, pl.ds(j * 16, 16)]
                     + c1 * rowbuf[e, pl.ds(GAT_OUT + j * 16, 16)]
                     + c2 * rowbuf[e, pl.ds(2 * GAT_OUT + j * 16, 16)]
                     + c3 * rowbuf[e, pl.ds(3 * GAT_OUT + j * 16, 16)])
                plsc.addupdate(acc.at[dloc, pl.ds(j * 16, 16)], v)
            return 0

        lax.fori_loop(0, cnt_k, edge, 0)
        return 0

    lax.fori_loop(0, nch, chunk, 0)
    pltpu.sync_copy(acc, out_h.at[pl.ds(base, NPT)])


_RB = 1024


def _mm_body(x_ref, w_ref, o_ref):
    o_ref[...] = jnp.dot(x_ref[...], w_ref[...],
                         preferred_element_type=jnp.float32)


def _mm(x, w):
    K = x.shape[1]
    Dout = w.shape[1]
    return pl.pallas_call(
        _mm_body,
        grid=(NPAD // _RB,),
        in_specs=[pl.BlockSpec((_RB, K), lambda i: (i, 0)),
                  pl.BlockSpec((K, Dout), lambda i: (0, 0))],
        out_specs=pl.BlockSpec((_RB, Dout), lambda i: (i, 0)),
        out_shape=jax.ShapeDtypeStruct((NPAD, Dout), jnp.float32),
    )(x, w)


def _stats_body(s_ref, o_ref):
    i = pl.program_id(0)
    b = s_ref[...]
    ssum = jnp.sum(b, axis=0, keepdims=True)
    ssq = jnp.sum(b * b, axis=0, keepdims=True)
    st = jnp.concatenate(
        [ssum, ssq, jnp.zeros((6, b.shape[1]), jnp.float32)], axis=0)

    @pl.when(i == 0)
    def _():
        o_ref[...] = st

    @pl.when(i != 0)
    def _():
        o_ref[...] += st


def _bn_stats(s):
    D = s.shape[1]
    return pl.pallas_call(
        _stats_body,
        grid=(NPAD // _RB,),
        in_specs=[pl.BlockSpec((_RB, D), lambda i: (i, 0))],
        out_specs=pl.BlockSpec((8, D), lambda i: (0, 0)),
        out_shape=jax.ShapeDtypeStruct((8, D), jnp.float32),
    )(s)


def _apply_body(s_ref, st_ref, g_ref, be_ref, o_ref):
    i = pl.program_id(0)
    b = s_ref[...]
    m = st_ref[0:1, :] * (1.0 / N)
    v = st_ref[1:2, :] * (1.0 / N) - m * m
    inv = lax.rsqrt(v + 1e-5)
    z = jax.nn.relu((b - m) * inv * g_ref[...] + be_ref[...])
    rows = i * _RB + lax.broadcasted_iota(jnp.int32, (_RB, 1), 0)
    o_ref[...] = jnp.where(rows < N, z, 0.0)


def _bn_apply_relu(s, st, g, be):
    D = s.shape[1]
    return pl.pallas_call(
        _apply_body,
        grid=(NPAD // _RB,),
        in_specs=[pl.BlockSpec((_RB, D), lambda i: (i, 0)),
                  pl.BlockSpec((8, D), lambda i: (0, 0)),
                  pl.BlockSpec((1, D), lambda i: (0, 0)),
                  pl.BlockSpec((1, D), lambda i: (0, 0))],
        out_specs=pl.BlockSpec((_RB, D), lambda i: (i, 0)),
        out_shape=jax.ShapeDtypeStruct((NPAD, D), jnp.float32),
    )(s, st, g.reshape(1, D), be.reshape(1, D))


def _dinv_body(d_ref, o_ref):
    d = d_ref[...]
    o_ref[...] = jnp.where(d > 0, lax.rsqrt(d), 0.0)


def _dinv(deg):
    return pl.pallas_call(
        _dinv_body,
        out_shape=jax.ShapeDtypeStruct((80, 128), jnp.float32),
    )(deg.reshape(80, 128)).reshape(NPAD)


def _pool_body(z_ref, b_ref, s_ref, c_ref):
    i = pl.program_id(0)
    zb = z_ref[...]
    bb = b_ref[...]
    oh = (bb == lax.broadcasted_iota(jnp.int32, (_RB, N_GRAPHS), 1)
          ).astype(jnp.float32)
    ps = lax.dot_general(oh, zb, (((0,), (0,)), ((), ())),
                         preferred_element_type=jnp.float32)
    pc = jnp.concatenate(
        [jnp.sum(oh, axis=0, keepdims=True),
         jnp.zeros((7, N_GRAPHS), jnp.float32)], axis=0)

    @pl.when(i == 0)
    def _():
        s_ref[...] = ps
        c_ref[...] = pc

    @pl.when(i != 0)
    def _():
        s_ref[...] += ps
        c_ref[...] += pc


def _pool(z, batch2d):
    return pl.pallas_call(
        _pool_body,
        grid=(NPAD // _RB,),
        in_specs=[pl.BlockSpec((_RB, GAT_OUT), lambda i: (i, 0)),
                  pl.BlockSpec((_RB, 1), lambda i: (i, 0))],
        out_specs=[pl.BlockSpec((N_GRAPHS, GAT_OUT), lambda i: (0, 0)),
                   pl.BlockSpec((8, N_GRAPHS), lambda i: (0, 0))],
        out_shape=[jax.ShapeDtypeStruct((N_GRAPHS, GAT_OUT), jnp.float32),
                   jax.ShapeDtypeStruct((8, N_GRAPHS), jnp.float32)],
    )(z, batch2d)


def _fc_body(s_ref, c_ref, w_ref, b_ref, o_ref):
    cnt = jnp.maximum(c_ref[0:1, :], 1.0)
    pooled = s_ref[...] / jnp.transpose(cnt)
    o_ref[...] = jax.nn.relu(
        jnp.dot(pooled, w_ref[...], preferred_element_type=jnp.float32)
        + b_ref[...])


def _fc(sums, cnts, w, b):
    return pl.pallas_call(
        _fc_body,
        out_shape=jax.ShapeDtypeStruct((N_GRAPHS, FUSED), jnp.float32),
    )(sums, cnts, w, b.reshape(1, FUSED))


def kernel(x, edge_index, batch, W1, b1, g1, be1, W2, b2, g2, be2,
           W3, b3, g3, be3, Wg, att_src, att_dst, bg, Wfc, bfc):
    loop = jnp.arange(N, dtype=jnp.int32)
    pad = EPAD - ETOT
    src = jnp.concatenate(
        [edge_index[0].astype(jnp.int32), loop,
         jnp.zeros((pad,), jnp.int32)])
    dst = jnp.concatenate(
        [edge_index[1].astype(jnp.int32), loop,
         jnp.full((pad,), SENTINEL, jnp.int32)])

    xp = jnp.concatenate(
        [x, jnp.zeros((NPAD - N, x.shape[1]), jnp.float32)], axis=0)
    batch2d = jnp.concatenate(
        [batch.astype(jnp.int32),
         jnp.full((NPAD - N,), N_GRAPHS, jnp.int32)]).reshape(NPAD, 1)

    deg, bsrc, bdst, cnts = _pre_kernel(src, dst)
    dinv = _dinv(deg)

    y = _mm(xp, W1)
    s = _gcn_scatter_128(y, dinv, bsrc, bdst, cnts)
    h = _bn_apply_relu(s, _bn_stats(s), g1, be1)
    y = _mm(h, W2)
    s = _gcn_scatter_256(y, dinv, bsrc, bdst, cnts)
    h = _bn_apply_relu(s, _bn_stats(s), g2, be2)
    y = _mm(h, W3)
    s = _gcn_scatter_256(y, dinv, bsrc, bdst, cnts)
    h = _bn_apply_relu(s, _bn_stats(s), g3, be3)

    hg = _mm(h, Wg)
    att_pad = jnp.concatenate(
        [jax.scipy.linalg.block_diag(*[att_src[i][:, None] for i in range(HEADS)]),
         jax.scipy.linalg.block_diag(*[att_dst[i][:, None] for i in range(HEADS)]),
         jnp.zeros((HEADS * GAT_OUT, 8), jnp.float32)], axis=1)
    asad = _mm(hg, att_pad)
    as_flat = asad[:, :HEADS].T.reshape(-1)
    ad_flat = asad[:, HEADS:2 * HEADS].T.reshape(-1)
    den, exb = _gat_a_kernel(as_flat, ad_flat, bsrc, bdst, cnts)
    gat = _gat_b_kernel(hg, den, exb, bsrc, bdst, cnts)
    h = _bn_apply_relu(gat, _bn_stats(gat), g3, be3)

    sums, cnt64 = _pool(h, batch2d)
    return _fc(sums, cnt64, Wfc, bfc)


# speedup vs baseline: 12.1309x; 1.5122x over previous
import functools

import jax
import jax.numpy as jnp
from jax import lax
from jax.experimental import pallas as pl
from jax.experimental.pallas import tpu as pltpu
from jax.experimental.pallas import tpu_sc as plsc

N = 10000
NPAD = 10240
NPT = 320
NTILES = 32
E = 320000
ETOT = E + N
ECHUNK = 2048
EPAD = ((ETOT + ECHUNK - 1) // ECHUNK) * ECHUNK
NCHUNKS = EPAD // ECHUNK
CAP = EPAD + 2 * ECHUNK
HEADS = 4
GAT_OUT = 256
N_GRAPHS = 64
FUSED = 512
SENTINEL = 1 << 20

_MESH = plsc.VectorSubcoreMesh(core_axis_name="c", subcore_axis_name="s")
_SC_PARAMS = pltpu.CompilerParams(needs_layout_passes=False)


def _wid_base():
    wid = lax.axis_index("s") * 2 + lax.axis_index("c")
    return wid, pl.multiple_of(wid * NPT, 8)


def _zero_rows(ref, nrows, ncols):
    z = jnp.zeros((16,), jnp.float32)

    def body(r, _):
        for j in range(ncols // 16):
            ref[r, pl.ds(j * 16, 16)] = z
        return 0

    lax.fori_loop(0, nrows, body, 0)


@functools.partial(
    pl.kernel,
    out_type=(
        jax.ShapeDtypeStruct((NPAD,), jnp.float32),
        jax.ShapeDtypeStruct((NTILES * CAP,), jnp.int32),
        jax.ShapeDtypeStruct((NTILES * CAP,), jnp.int32),
        jax.ShapeDtypeStruct((NTILES * 16,), jnp.int32),
    ),
    mesh=_MESH,
    compiler_params=_SC_PARAMS,
    scratch_types=[
        pltpu.VMEM((ECHUNK,), jnp.int32),
        pltpu.VMEM((ECHUNK,), jnp.int32),
        pltpu.VMEM((ECHUNK,), jnp.int32),
        pltpu.VMEM((ECHUNK,), jnp.int32),
        pltpu.VMEM((2 * ECHUNK + 16,), jnp.int32),
        pltpu.VMEM((2 * ECHUNK + 16,), jnp.int32),
        pltpu.VMEM((NPT,), jnp.float32),
        pltpu.VMEM((16,), jnp.int32),
        pltpu.SemaphoreType.DMA,
        pltpu.SemaphoreType.DMA,
        pltpu.SemaphoreType.DMA,
        pltpu.SemaphoreType.DMA,
    ],
)
def _pre_kernel(src_h, dst_h, deg_h, bsrc_h, bdst_h, cnt_h,
                esrc0, esrc1, edst0, edst1, osrc, odst, degl, splat,
                ses0, ses1, sed0, sed1):
    wid, base = _wid_base()
    esrc = (esrc0, esrc1)
    edst = (edst0, edst1)
    ses = (ses0, ses1)
    sed = (sed0, sed1)
    izero = jnp.zeros((16,), jnp.int32)

    def zi(ref, nwords):
        def body(i, _):
            ref[pl.ds(i * 16, 16)] = izero
            return 0
        lax.fori_loop(0, nwords // 16, body, 0)

    zi(osrc, 2 * ECHUNK + 16)
    zi(odst, 2 * ECHUNK + 16)
    zf = jnp.zeros((16,), jnp.float32)
    for r in range(NPT // 16):
        degl[pl.ds(r * 16, 16)] = zf

    def eoff(k):
        kk = jnp.clip(k, 0, NCHUNKS - 1)
        return pl.multiple_of(kk * ECHUNK, 8)

    def issue(k, b):
        pltpu.async_copy(src_h.at[pl.ds(eoff(k), ECHUNK)], esrc[b], ses[b])
        pltpu.async_copy(dst_h.at[pl.ds(eoff(k), ECHUNK)], edst[b], sed[b])

    def wait(k, b):
        pltpu.make_async_copy(src_h.at[pl.ds(eoff(k), ECHUNK)], esrc[b], ses[b]).wait()
        pltpu.make_async_copy(dst_h.at[pl.ds(eoff(k), ECHUNK)], edst[b], sed[b]).wait()

    ones = jnp.ones((16,), jnp.float32)

    issue(0, 0)

    def pair(g, carry):
        for b in (0, 1):
            k = 2 * g + b
            wait(k, b)
            issue(k + 1, 1 - b)
            cursor, out_off = carry

            def inner(i, cur):
                sv = esrc[b][pl.ds(i * 16, 16)]
                dv = edst[b][pl.ds(i * 16, 16)]
                mask = (dv >= base) & (dv < base + NPT)
                dloc = jnp.where(mask, dv - base, 0)
                plsc.addupdate_scatter(degl, [dloc], ones, mask=mask)
                plsc.store_compressed(osrc.at[pl.ds(cur, 16)], sv, mask=mask)
                plsc.store_compressed(odst.at[pl.ds(cur, 16)], dloc, mask=mask)
                return cur + jnp.sum(jnp.where(mask, 1, 0))

            cursor = lax.fori_loop(0, ECHUNK // 16, inner, cursor)

            do_flush = cursor >= ECHUNK

            @pl.when(do_flush)
            def _():
                dsthb = pl.multiple_of(wid * CAP + out_off, 8)
                pltpu.sync_copy(osrc.at[pl.ds(0, ECHUNK)],
                                bsrc_h.at[pl.ds(dsthb, ECHUNK)])
                pltpu.sync_copy(odst.at[pl.ds(0, ECHUNK)],
                                bdst_h.at[pl.ds(dsthb, ECHUNK)])
                for r in range(ECHUNK // 16 + 1):
                    osrc[pl.ds(r * 16, 16)] = osrc[pl.ds(ECHUNK + r * 16, 16)]
                    odst[pl.ds(r * 16, 16)] = odst[pl.ds(ECHUNK + r * 16, 16)]

            shift = jnp.where(do_flush, ECHUNK, 0)
            carry = (cursor - shift, out_off + shift)
        return carry

    cursor, out_off = lax.fori_loop(0, NCHUNKS // 2, pair,
                                    (jnp.int32(0), jnp.int32(0)))
    wait(NCHUNKS, 0)

    dsthb = pl.multiple_of(wid * CAP + out_off, 8)
    pltpu.sync_copy(osrc.at[pl.ds(0, ECHUNK)], bsrc_h.at[pl.ds(dsthb, ECHUNK)])
    pltpu.sync_copy(odst.at[pl.ds(0, ECHUNK)], bdst_h.at[pl.ds(dsthb, ECHUNK)])
    pltpu.sync_copy(osrc.at[pl.ds(ECHUNK, ECHUNK)],
                    bsrc_h.at[pl.ds(dsthb + ECHUNK, ECHUNK)])
    pltpu.sync_copy(odst.at[pl.ds(ECHUNK, ECHUNK)],
                    bdst_h.at[pl.ds(dsthb + ECHUNK, ECHUNK)])

    pltpu.sync_copy(degl, deg_h.at[pl.ds(base, NPT)])
    total = out_off + cursor
    splat[pl.ds(0, 16)] = jnp.broadcast_to(total, (16,)).astype(jnp.int32)
    pltpu.sync_copy(splat, cnt_h.at[pl.ds(pl.multiple_of(wid * 16, 8), 16)])


def _make_gcn_scatter(D, C=64):
    @functools.partial(
        pl.kernel,
        out_type=jax.ShapeDtypeStruct((NPAD, D), jnp.float32),
        mesh=_MESH,
        compiler_params=_SC_PARAMS,
        scratch_types=[
            pltpu.VMEM((C,), jnp.int32),
            pltpu.VMEM((C,), jnp.int32),
            pltpu.VMEM((C + 16,), jnp.int32),
            pltpu.VMEM((C + 16,), jnp.int32),
            pltpu.VMEM((C, D), jnp.float32),
            pltpu.VMEM((C, D), jnp.float32),
            pltpu.VMEM((NPT, D), jnp.float32),
            pltpu.VMEM((16,), jnp.int32),
            pltpu.SemaphoreType.DMA,
            pltpu.SemaphoreType.DMA,
            pltpu.SemaphoreType.DMA,
            pltpu.SemaphoreType.DMA,
        ],
    )
    def gcn_k(y_h, bsrc_h, bdst_h, cnt_h, out_h,
              srcb0, srcb1, dstb0, dstb1, rb0, rb1, acc, cntv,
              si0, si1, sg0, sg1):
        wid, base = _wid_base()
        srcb = (srcb0, srcb1)
        dstb = (dstb0, dstb1)
        rb = (rb0, rb1)
        si = (si0, si1)
        sg = (sg0, sg1)
        pltpu.sync_copy(cnt_h.at[pl.ds(pl.multiple_of(wid * 16, 8), 16)], cntv)
        c = cntv[pl.ds(0, 16)][0]
        _zero_rows(acc, NPT, D)

        nch = (c + (C - 1)) // C

        def koff(k):
            kk = jnp.clip(k, 0, jnp.maximum(nch - 1, 0))
            return pl.multiple_of(wid * CAP + kk * C, 8)

        def issue_idx(k, b):
            pltpu.async_copy(bsrc_h.at[pl.ds(koff(k), C)], srcb[b], si[b])
            pltpu.async_copy(bdst_h.at[pl.ds(koff(k), C)],
                             dstb[b].at[pl.ds(0, C)], si[b])

        def wait_idx(k, b):
            pltpu.make_async_copy(bsrc_h.at[pl.ds(koff(k), C)],
                                  srcb[b], si[b]).wait()
            pltpu.make_async_copy(bdst_h.at[pl.ds(koff(k), C)],
                                  dstb[b].at[pl.ds(0, C)], si[b]).wait()

        def issue_gather(b):
            pltpu.async_copy(y_h.at[srcb[b]], rb[b], sg[b])

        def wait_gather(b):
            pltpu.make_async_copy(y_h.at[srcb[b]], rb[b], sg[b]).wait()

        issue_idx(0, 0)
        wait_idx(0, 0)
        issue_gather(0)
        issue_idx(1, 1)

        def pair(g, _):
            for b in (0, 1):
                k = 2 * g + b
                wait_gather(b)
                wait_idx(k + 1, 1 - b)
                issue_gather(1 - b)

                cnt_k = jnp.clip(c - k * C, 0, C)

                def edge(e, _):
                    dloc = dstb[b][pl.ds(e, 16)][0]
                    for j in range(D // 16):
                        plsc.addupdate(acc.at[dloc, pl.ds(j * 16, 16)],
                                       rb[b][e, pl.ds(j * 16, 16)])
                    return 0

                lax.fori_loop(0, cnt_k, edge, 0)
                issue_idx(k + 2, b)
            return 0

        npairs = (nch + 1) // 2
        lax.fori_loop(0, npairs, pair, 0)
        wait_gather(0)
        wait_idx(2 * npairs + 1, 1)
        pltpu.sync_copy(acc, out_h.at[pl.ds(base, NPT)])

    return gcn_k


_gcn_scatter_128 = _make_gcn_scatter(128)
_gcn_scatter_256 = _make_gcn_scatter(256)


CA = 512


@functools.partial(
    pl.kernel,
    out_type=(
        jax.ShapeDtypeStruct((HEADS * NPAD,), jnp.float32),
        jax.ShapeDtypeStruct((NTILES * CAP * HEADS,), jnp.float32),
    ),
    mesh=_MESH,
    compiler_params=_SC_PARAMS,
    scratch_types=[
        pltpu.VMEM((HEADS * NPAD,), jnp.float32),
        pltpu.VMEM((HEADS * NPT,), jnp.float32),
        pltpu.VMEM((HEADS * NPT,), jnp.float32),
        pltpu.VMEM((HEADS * CA,), jnp.float32),
        pltpu.VMEM((CA,), jnp.int32),
        pltpu.VMEM((CA,), jnp.int32),
        pltpu.VMEM((16,), jnp.int32),
    ],
)
def _gat_a_kernel(as_h, ad_h, bsrc_h, bdst_h, cnt_h, den_h, exb_h,
                  asv, adl, denl, exst, srcb, dstb, cntv):
    wid, base = _wid_base()
    pltpu.sync_copy(cnt_h.at[pl.ds(pl.multiple_of(wid * 16, 8), 16)], cntv)
    c = cntv[pl.ds(0, 16)][0]
    pltpu.sync_copy(as_h, asv)
    for h in range(HEADS):
        pltpu.sync_copy(ad_h.at[pl.ds(pl.multiple_of(h * NPAD + base, 8), NPT)],
                        adl.at[pl.ds(h * NPT, NPT)])
    zf = jnp.zeros((16,), jnp.float32)
    for r in range(HEADS * NPT // 16):
        denl[pl.ds(r * 16, 16)] = zf

    lanes = lax.iota(jnp.int32, 16)
    nch = (c + (CA - 1)) // CA

    def chunk(k, _):
        off = pl.multiple_of(wid * CAP + jnp.clip(k, 0, None) * CA, 8)
        pltpu.sync_copy(bsrc_h.at[pl.ds(off, CA)], srcb)
        pltpu.sync_copy(bdst_h.at[pl.ds(off, CA)], dstb)
        for i in range(CA // 16):
            sv = srcb[pl.ds(i * 16, 16)]
            dlv = dstb[pl.ds(i * 16, 16)]
            mask = (k * CA + i * 16 + lanes) < c
            pos = (i * 16 + lanes) * HEADS
            for h in range(HEADS):
                a1 = plsc.load_gather(asv, [sv + h * NPAD])
                a2 = plsc.load_gather(adl, [dlv + h * NPT])
                ev = a1 + a2
                ev = jnp.maximum(ev, 0.2 * ev)
                ex = jnp.exp(ev)
                plsc.addupdate_scatter(denl, [dlv + h * NPT], ex, mask=mask)
                plsc.store_scatter(exst, [pos + h], ex)
        pltpu.sync_copy(
            exst,
            exb_h.at[pl.ds(pl.multiple_of((wid * CAP + k * CA) * HEADS, 8),
                           HEADS * CA)])
        return 0

    lax.fori_loop(0, nch, chunk, 0)
    for h in range(HEADS):
        pltpu.sync_copy(denl.at[pl.ds(h * NPT, NPT)],
                        den_h.at[pl.ds(pl.multiple_of(h * NPAD + base, 8), NPT)])


CB = 16


@functools.partial(
    pl.kernel,
    out_type=jax.ShapeDtypeStruct((NPAD, GAT_OUT), jnp.float32),
    mesh=_MESH,
    compiler_params=_SC_PARAMS,
    scratch_types=[
        pltpu.VMEM((HEADS * NPT,), jnp.float32),
        pltpu.VMEM((NPT, GAT_OUT), jnp.float32),
        pltpu.VMEM((CB, HEADS * GAT_OUT), jnp.float32),
        pltpu.VMEM((CB, HEADS * GAT_OUT), jnp.float32),
        pltpu.VMEM((HEADS * CB,), jnp.float32),
        pltpu.VMEM((HEADS * CB,), jnp.float32),
        pltpu.VMEM((HEADS * CB + 16,), jnp.float32),
        pltpu.VMEM((CB,), jnp.int32),
        pltpu.VMEM((CB,), jnp.int32),
        pltpu.VMEM((CB + 16,), jnp.int32),
        pltpu.VMEM((CB + 16,), jnp.int32),
        pltpu.VMEM((16,), jnp.int32),
        pltpu.SemaphoreType.DMA,
        pltpu.SemaphoreType.DMA,
        pltpu.SemaphoreType.DMA,
        pltpu.SemaphoreType.DMA,
    ],
)
def _gat_b_kernel(hg_h, den_h, exb_h, bsrc_h, bdst_h, cnt_h, out_h,
                  denl, acc, rb0, rb1, exc0, exc1, coefb,
                  srcb0, srcb1, dstb0, dstb1, cntv, si0, si1, sg0, sg1):
    wid, base = _wid_base()
    srcb = (srcb0, srcb1)
    dstb = (dstb0, dstb1)
    rb = (rb0, rb1)
    exc = (exc0, exc1)
    si = (si0, si1)
    sg = (sg0, sg1)
    pltpu.sync_copy(cnt_h.at[pl.ds(pl.multiple_of(wid * 16, 8), 16)], cntv)
    c = cntv[pl.ds(0, 16)][0]
    for h in range(HEADS):
        pltpu.sync_copy(den_h.at[pl.ds(pl.multiple_of(h * NPAD + base, 8), NPT)],
                        denl.at[pl.ds(h * NPT, NPT)])
    _zero_rows(acc, NPT, GAT_OUT)

    lanes = lax.iota(jnp.int32, 16)
    nch = (c + (CB - 1)) // CB

    def koff(k):
        kk = jnp.clip(k, 0, jnp.maximum(nch - 1, 0))
        return pl.multiple_of(wid * CAP + kk * CB, 8)

    def xoff(k):
        kk = jnp.clip(k, 0, jnp.maximum(nch - 1, 0))
        return pl.multiple_of((wid * CAP + kk * CB) * HEADS, 8)

    def issue_idx(k, b):
        pltpu.async_copy(bsrc_h.at[pl.ds(koff(k), CB)], srcb[b], si[b])
        pltpu.async_copy(bdst_h.at[pl.ds(koff(k), CB)],
                         dstb[b].at[pl.ds(0, CB)], si[b])
        pltpu.async_copy(exb_h.at[pl.ds(xoff(k), HEADS * CB)], exc[b], si[b])

    def wait_idx(k, b):
        pltpu.make_async_copy(bsrc_h.at[pl.ds(koff(k), CB)],
                              srcb[b], si[b]).wait()
        pltpu.make_async_copy(bdst_h.at[pl.ds(koff(k), CB)],
                              dstb[b].at[pl.ds(0, CB)], si[b]).wait()
        pltpu.make_async_copy(exb_h.at[pl.ds(xoff(k), HEADS * CB)],
                              exc[b], si[b]).wait()

    def issue_gather(b):
        pltpu.async_copy(hg_h.at[srcb[b]], rb[b], sg[b])

    def wait_gather(b):
        pltpu.make_async_copy(hg_h.at[srcb[b]], rb[b], sg[b]).wait()

    issue_idx(0, 0)
    wait_idx(0, 0)
    issue_gather(0)
    issue_idx(1, 1)

    def pair(g, _):
        for b in (0, 1):
            k = 2 * g + b
            wait_gather(b)
            wait_idx(k + 1, 1 - b)
            issue_gather(1 - b)

            dlv = dstb[b][pl.ds(0, 16)]
            for h in range(HEADS):
                dg = plsc.load_gather(denl, [dlv + h * NPT])
                ev = plsc.load_gather(exc[b], [lanes * HEADS + h])
                coefb[pl.ds(h * CB, 16)] = ev / (dg + 1e-16) * 0.25

            cnt_k = jnp.clip(c - k * CB, 0, CB)

            def edge(e, _):
                dloc = dstb[b][pl.ds(e, 16)][0]
                c0 = coefb[pl.ds(e, 16)][0]
                c1 = coefb[pl.ds(CB + e, 16)][0]
                c2 = coefb[pl.ds(2 * CB + e, 16)][0]
                c3 = coefb[pl.ds(3 * CB + e, 16)][0]
                for j in range(GAT_OUT // 16):
                    v = (c0 * rb[b][e, pl.ds(j * 16, 16)]
                         + c1 * rb[b][e, pl.ds(GAT_OUT + j * 16, 16)]
                         + c2 * rb[b][e, pl.ds(2 * GAT_OUT + j * 16, 16)]
                         + c3 * rb[b][e, pl.ds(3 * GAT_OUT + j * 16, 16)])
                    plsc.addupdate(acc.at[dloc, pl.ds(j * 16, 16)], v)
                return 0

            lax.fori_loop(0, cnt_k, edge, 0)
            issue_idx(k + 2, b)
        return 0

    npairs = (nch + 1) // 2
    lax.fori_loop(0, npairs, pair, 0)
    wait_gather(0)
    wait_idx(2 * npairs + 1, 1)
    pltpu.sync_copy(acc, out_h.at[pl.ds(base, NPT)])


_RB = 1024


def _mm_body(x_ref, w_ref, o_ref):
    o_ref[...] = jnp.dot(x_ref[...], w_ref[...],
                         preferred_element_type=jnp.float32)


def _mm_scaled_body(x_ref, w_ref, d_ref, o_ref):
    o_ref[...] = jnp.dot(x_ref[...], w_ref[...],
                         preferred_element_type=jnp.float32) * d_ref[...]


def _mm(x, w, rowscale=None):
    K = x.shape[1]
    Dout = w.shape[1]
    if rowscale is None:
        return pl.pallas_call(
            _mm_body,
            grid=(NPAD // _RB,),
            in_specs=[pl.BlockSpec((_RB, K), lambda i: (i, 0)),
                      pl.BlockSpec((K, Dout), lambda i: (0, 0))],
            out_specs=pl.BlockSpec((_RB, Dout), lambda i: (i, 0)),
            out_shape=jax.ShapeDtypeStruct((NPAD, Dout), jnp.float32),
        )(x, w)
    return pl.pallas_call(
        _mm_scaled_body,
        grid=(NPAD // _RB,),
        in_specs=[pl.BlockSpec((_RB, K), lambda i: (i, 0)),
                  pl.BlockSpec((K, Dout), lambda i: (0, 0)),
                  pl.BlockSpec((_RB, 1), lambda i: (i, 0))],
        out_specs=pl.BlockSpec((_RB, Dout), lambda i: (i, 0)),
        out_shape=jax.ShapeDtypeStruct((NPAD, Dout), jnp.float32),
    )(x, w, rowscale)


def _stats_body(s_ref, d_ref, o_ref):
    i = pl.program_id(0)
    b = s_ref[...] * d_ref[...]
    ssum = jnp.sum(b, axis=0, keepdims=True)
    ssq = jnp.sum(b * b, axis=0, keepdims=True)
    st = jnp.concatenate(
        [ssum, ssq, jnp.zeros((6, b.shape[1]), jnp.float32)], axis=0)

    @pl.when(i == 0)
    def _():
        o_ref[...] = st

    @pl.when(i != 0)
    def _():
        o_ref[...] += st


def _bn_stats(s, rowscale):
    D = s.shape[1]
    return pl.pallas_call(
        _stats_body,
        grid=(NPAD // _RB,),
        in_specs=[pl.BlockSpec((_RB, D), lambda i: (i, 0)),
                  pl.BlockSpec((_RB, 1), lambda i: (i, 0))],
        out_specs=pl.BlockSpec((8, D), lambda i: (0, 0)),
        out_shape=jax.ShapeDtypeStruct((8, D), jnp.float32),
    )(s, rowscale)


def _apply_body(s_ref, d_ref, st_ref, g_ref, be_ref, o_ref):
    i = pl.program_id(0)
    b = s_ref[...] * d_ref[...]
    m = st_ref[0:1, :] * (1.0 / N)
    v = st_ref[1:2, :] * (1.0 / N) - m * m
    inv = lax.rsqrt(v + 1e-5)
    z = jax.nn.relu((b - m) * inv * g_ref[...] + be_ref[...])
    rows = i * _RB + lax.broadcasted_iota(jnp.int32, (_RB, 1), 0)
    o_ref[...] = jnp.where(rows < N, z, 0.0)


def _bn_apply_relu(s, st, g, be, rowscale):
    D = s.shape[1]
    return pl.pallas_call(
        _apply_body,
        grid=(NPAD // _RB,),
        in_specs=[pl.BlockSpec((_RB, D), lambda i: (i, 0)),
                  pl.BlockSpec((_RB, 1), lambda i: (i, 0)),
                  pl.BlockSpec((8, D), lambda i: (0, 0)),
                  pl.BlockSpec((1, D), lambda i: (0, 0)),
                  pl.BlockSpec((1, D), lambda i: (0, 0))],
        out_specs=pl.BlockSpec((_RB, D), lambda i: (i, 0)),
        out_shape=jax.ShapeDtypeStruct((NPAD, D), jnp.float32),
    )(s, rowscale, st, g.reshape(1, D), be.reshape(1, D))


def _dinv_body(d_ref, o_ref):
    d = d_ref[...]
    o_ref[...] = jnp.where(d > 0, lax.rsqrt(d), 0.0)


def _dinv(deg):
    return pl.pallas_call(
        _dinv_body,
        out_shape=jax.ShapeDtypeStruct((80, 128), jnp.float32),
    )(deg.reshape(80, 128)).reshape(NPAD)


def _pool_body(z_ref, b_ref, s_ref, c_ref):
    i = pl.program_id(0)
    zb = z_ref[...]
    bb = b_ref[...]
    oh = (bb == lax.broadcasted_iota(jnp.int32, (_RB, N_GRAPHS), 1)
          ).astype(jnp.float32)
    ps = lax.dot_general(oh, zb, (((0,), (0,)), ((), ())),
                         preferred_element_type=jnp.float32)
    pc = jnp.concatenate(
        [jnp.sum(oh, axis=0, keepdims=True),
         jnp.zeros((7, N_GRAPHS), jnp.float32)], axis=0)

    @pl.when(i == 0)
    def _():
        s_ref[...] = ps
        c_ref[...] = pc

    @pl.when(i != 0)
    def _():
        s_ref[...] += ps
        c_ref[...] += pc


def _pool(z, batch2d):
    return pl.pallas_call(
        _pool_body,
        grid=(NPAD // _RB,),
        in_specs=[pl.BlockSpec((_RB, GAT_OUT), lambda i: (i, 0)),
                  pl.BlockSpec((_RB, 1), lambda i: (i, 0))],
        out_specs=[pl.BlockSpec((N_GRAPHS, GAT_OUT), lambda i: (0, 0)),
                   pl.BlockSpec((8, N_GRAPHS), lambda i: (0, 0))],
        out_shape=[jax.ShapeDtypeStruct((N_GRAPHS, GAT_OUT), jnp.float32),
                   jax.ShapeDtypeStruct((8, N_GRAPHS), jnp.float32)],
    )(z, batch2d)


def _fc_body(s_ref, c_ref, w_ref, b_ref, o_ref):
    cnt = jnp.maximum(c_ref[0:1, :], 1.0)
    pooled = s_ref[...] / jnp.transpose(cnt)
    o_ref[...] = jax.nn.relu(
        jnp.dot(pooled, w_ref[...], preferred_element_type=jnp.float32)
        + b_ref[...])


def _fc(sums, cnts, w, b):
    return pl.pallas_call(
        _fc_body,
        out_shape=jax.ShapeDtypeStruct((N_GRAPHS, FUSED), jnp.float32),
    )(sums, cnts, w, b.reshape(1, FUSED))


def kernel(x, edge_index, batch, W1, b1, g1, be1, W2, b2, g2, be2,
           W3, b3, g3, be3, Wg, att_src, att_dst, bg, Wfc, bfc):
    loop = jnp.arange(N, dtype=jnp.int32)
    pad = EPAD - ETOT
    src = jnp.concatenate(
        [edge_index[0].astype(jnp.int32), loop,
         jnp.zeros((pad,), jnp.int32)])
    dst = jnp.concatenate(
        [edge_index[1].astype(jnp.int32), loop,
         jnp.full((pad,), SENTINEL, jnp.int32)])

    xp = jnp.concatenate(
        [x, jnp.zeros((NPAD - N, x.shape[1]), jnp.float32)], axis=0)
    batch2d = jnp.concatenate(
        [batch.astype(jnp.int32),
         jnp.full((NPAD - N,), N_GRAPHS, jnp.int32)]).reshape(NPAD, 1)

    deg, bsrc, bdst, cnts = _pre_kernel(src, dst)
    dinv2d = _dinv(deg).reshape(NPAD, 1)

    h = xp
    for W, g_, be_, scat in ((W1, g1, be1, _gcn_scatter_128),
                             (W2, g2, be2, _gcn_scatter_256),
                             (W3, g3, be3, _gcn_scatter_256)):
        y = _mm(h, W, rowscale=dinv2d)
        s = scat(y, bsrc, bdst, cnts)
        h = _bn_apply_relu(s, _bn_stats(s, dinv2d), g_, be_, dinv2d)

    ones2d = jnp.ones((NPAD, 1), jnp.float32)
    hg = _mm(h, Wg)
    att_pad = jnp.concatenate(
        [jax.scipy.linalg.block_diag(*[att_src[i][:, None] for i in range(HEADS)]),
         jax.scipy.linalg.block_diag(*[att_dst[i][:, None] for i in range(HEADS)]),
         jnp.zeros((HEADS * GAT_OUT, 8), jnp.float32)], axis=1)
    asad = _mm(hg, att_pad)
    as_flat = asad[:, :HEADS].T.reshape(-1)
    ad_flat = asad[:, HEADS:2 * HEADS].T.reshape(-1)
    den, exb = _gat_a_kernel(as_flat, ad_flat, bsrc, bdst, cnts)
    gat = _gat_b_kernel(hg, den, exb, bsrc, bdst, cnts)
    h = _bn_apply_relu(gat, _bn_stats(gat, ones2d), g3, be3, ones2d)

    sums, cnt64 = _pool(h, batch2d)
    return _fc(sums, cnt64, Wfc, bfc)
